# baseline (speedup 1.0000x reference)
import functools

import jax
import jax.numpy as jnp
from jax import lax
from jax.experimental import pallas as pl
from jax.experimental.pallas import tpu as pltpu
from jax.experimental.pallas import tpu_sc as plsc

_NC = 2
_NS = 16
_NW = _NC * _NS
_SUB = 128
_G = 4


def _pick_g(per_w):
    for g in (4, 2):
        if per_w % g == 0 and (per_w // g) % 2 == 0:
            return g, True
    for g in (8, 5, 4, 2, 1):
        if per_w % g == 0:
            return g, False
    raise ValueError(per_w)


def _sc_gather3(idx0, idx1, idx2, tab0, tab1, tab2):
    n_sub = idx0.shape[0]
    d0, d1, d2 = tab0.shape[1], tab1.shape[1], tab2.shape[1]
    per_w = n_sub // _NW
    _G, pipelined = _pick_g(per_w)
    n_grp = per_w // _G
    n_pair = n_grp // 2
    n_sets = 2 if pipelined else 1
    mesh = plsc.VectorSubcoreMesh(core_axis_name="c", subcore_axis_name="s")

    buf_set = [
        pltpu.VMEM((_G, _SUB), jnp.int32),
        pltpu.VMEM((_G, _SUB), jnp.int32),
        pltpu.VMEM((_G, _SUB), jnp.int32),
        pltpu.VMEM((_G * _SUB, d0), tab0.dtype),
        pltpu.VMEM((_G * _SUB, d1), tab1.dtype),
        pltpu.VMEM((_G * _SUB, d2), tab2.dtype),
        pltpu.SemaphoreType.DMA,
        pltpu.SemaphoreType.DMA,
        pltpu.SemaphoreType.DMA,
    ]

    @functools.partial(
        pl.kernel,
        out_type=jax.ShapeDtypeStruct((n_sub * _SUB, 128), tab0.dtype),
        mesh=mesh,
        scratch_types=buf_set * n_sets,
        compiler_params=pltpu.CompilerParams(use_tc_tiling_on_sc=False),
    )
    def k(i0_hbm, i1_hbm, i2_hbm, t0_hbm, t1_hbm, t2_hbm, o_hbm, *scratch):
        set_a, set_b = scratch[:9], scratch[9:] if pipelined else scratch[:9]
        wid = lax.axis_index("s") * _NC + lax.axis_index("c")
        tabs = (t0_hbm, t1_hbm, t2_hbm)
        idx_hbms = (i0_hbm, i1_hbm, i2_hbm)
        lane_off = (0, d0, d0 + d1)

        def s1(g, S):
            ivs, rvs, isem, gsem = S[0:3], S[3:6], S[6], S[7]
            base_sub = wid * per_w + g * _G
            for c in [pltpu.async_copy(ih.at[pl.ds(base_sub, _G)], iv, isem)
                      for ih, iv in zip(idx_hbms, ivs)]:
                c.wait()
            for j in range(_G):
                sl = pl.ds(j * _SUB, _SUB)
                for t, iv, rv in zip(tabs, ivs, rvs):
                    pltpu.async_copy(t.at[iv.at[j]], rv.at[sl], gsem)

        def s2(g, S):
            ivs, rvs, gsem, wsem = S[0:3], S[3:6], S[7], S[8]
            for j in range(_G):
                sl = pl.ds(j * _SUB, _SUB)
                for t, iv, rv in zip(tabs, ivs, rvs):
                    pltpu.make_async_copy(t.at[iv.at[j]], rv.at[sl], gsem).wait()
            rows = pl.ds((wid * per_w + g * _G) * _SUB, _G * _SUB)
            for rv, off, d in zip(rvs, lane_off, (d0, d1, d2)):
                pltpu.async_copy(rv, o_hbm.at[rows, pl.ds(off, d)], wsem)

        def s3(S):
            rvs, wsem = S[3:6], S[8]
            rows = pl.ds(0, _G * _SUB)
            for rv, off, d in zip(rvs, lane_off, (d0, d1, d2)):
                pltpu.make_async_copy(rv, o_hbm.at[rows, pl.ds(off, d)], wsem).wait()

        if pipelined:
            s1(0, set_a)
            s1(1, set_b)

            def body(p, carry):
                g = 2 * p
                s2(g, set_a)
                s2(g + 1, set_b)
                s3(set_a)
                s1(g + 2, set_a)
                s3(set_b)
                s1(g + 3, set_b)
                return carry

            lax.fori_loop(0, n_pair - 1, body, 0)
            s2(n_grp - 2, set_a)
            s2(n_grp - 1, set_b)
            s3(set_a)
            s3(set_b)
        else:
            def body(g, carry):
                s1(g, set_a)
                s2(g, set_a)
                s3(set_a)
                return carry

            lax.fori_loop(0, n_grp, body, 0)

    return k(idx0, idx1, idx2, tab0, tab1, tab2)


def _tc_body(comb_ref, w_ref, b_ref, o_ref, *, d_valid):
    x = comb_ref[...]
    lanes = lax.broadcasted_iota(jnp.int32, x.shape, 1)
    x = jnp.where(lanes < d_valid, x, 0.0)
    o_ref[...] = jnp.dot(x, w_ref[...],
                         preferred_element_type=jnp.float32) + b_ref[...]


def _tc_chunk_body(comb_ref, w_ref, b_ref, o_ref, *, d_valid):
    _tc_body(comb_ref, w_ref, b_ref, o_ref, d_valid=d_valid)


def _tc_chunk_body_alias(prev_ref, comb_ref, w_ref, b_ref, o_ref, *, d_valid):
    del prev_ref
    _tc_body(comb_ref, w_ref, b_ref, o_ref, d_valid=d_valid)


def _tc_combine_chunk(comb, w_pad, b, d_valid, n, chunk, prev_out):
    n_c = comb.shape[0]
    bs = 16384
    while n_c % bs:
        bs //= 2
    spc = n_c // bs
    blk = lambda i: (i, 0)
    out_blk = lambda i: (i + chunk * spc, 0)
    full = lambda i: (0, 0)
    d_out = w_pad.shape[1]
    out_shape = jax.ShapeDtypeStruct((n, d_out), jnp.float32)
    specs = [
        pl.BlockSpec((bs, comb.shape[1]), blk),
        pl.BlockSpec(w_pad.shape, full),
        pl.BlockSpec((1, d_out), full),
    ]
    body = functools.partial(_tc_body, d_valid=d_valid)
    if prev_out is None:
        return pl.pallas_call(
            body,
            grid=(spc,),
            in_specs=specs,
            out_specs=pl.BlockSpec((bs, d_out), out_blk),
            out_shape=out_shape,
        )(comb, w_pad, b)
    return pl.pallas_call(
        functools.partial(_tc_chunk_body_alias, d_valid=d_valid),
        grid=(spc,),
        in_specs=[pl.BlockSpec((8, d_out), full)] + specs,
        out_specs=pl.BlockSpec((bs, d_out), out_blk),
        out_shape=out_shape,
        input_output_aliases={0: 0},
    )(prev_out, comb, w_pad, b)


def kernel(card_ids, mana_costs, card_types, powers, toughnesses,
           card_table, mana_table, type_table, power_table, tough_table, W, b):
    bsz, seq = card_ids.shape
    n = bsz * seq
    n_mana = mana_table.shape[0]
    n_type = type_table.shape[0]
    n_pow = power_table.shape[0]
    n_tgh = tough_table.shape[0]

    mt_tab = jnp.concatenate(
        [jnp.repeat(mana_table, n_type, axis=0), jnp.tile(type_table, (n_mana, 1))],
        axis=1)
    pt_tab = jnp.concatenate(
        [jnp.repeat(power_table, n_tgh, axis=0), jnp.tile(tough_table, (n_pow, 1))],
        axis=1)

    to2d = lambda a: a.reshape(n // _SUB, _SUB).astype(jnp.int32)
    card_idx = to2d(card_ids)
    mt_idx = to2d(mana_costs * n_type + card_types)
    pt_idx = to2d(powers * n_tgh + toughnesses)

    w_pad = jnp.concatenate(
        [W, jnp.zeros((128 - W.shape[0], W.shape[1]), W.dtype)], axis=0)
    b2 = b.reshape(1, -1)

    n_chunks = 4
    n_sub_c = (n // _SUB) // n_chunks
    out = None
    for c in range(n_chunks):
        sl = slice(c * n_sub_c, (c + 1) * n_sub_c)
        comb_c = _sc_gather3(card_idx[sl], mt_idx[sl], pt_idx[sl],
                             card_table, mt_tab, pt_tab)
        out = _tc_combine_chunk(comb_c, w_pad, b2, W.shape[0], n, c, out)
    return out.reshape(bsz, seq, W.shape[1])


# speedup vs baseline: 1.0849x; 1.0849x over previous
import functools

import jax
import jax.numpy as jnp
from jax import lax
from jax.experimental import pallas as pl
from jax.experimental.pallas import tpu as pltpu
from jax.experimental.pallas import tpu_sc as plsc

_NC = 2
_NS = 16
_NW = _NC * _NS
_SUB = 128
_G = 4


def _pick_g(per_w):
    for g in (4, 2):
        if per_w % g == 0 and (per_w // g) % 2 == 0:
            return g, True
    for g in (8, 5, 4, 2, 1):
        if per_w % g == 0:
            return g, False
    raise ValueError(per_w)


def _sc_gather3(idx0, idx1, idx2, tab0, tab1, tab2):
    n_sub = idx0.shape[0]
    d0, d1, d2 = tab0.shape[1], tab1.shape[1], tab2.shape[1]
    per_w = n_sub // _NW
    _G, pipelined = _pick_g(per_w)
    n_grp = per_w // _G
    n_pair = n_grp // 2
    n_sets = 2 if pipelined else 1
    mesh = plsc.VectorSubcoreMesh(core_axis_name="c", subcore_axis_name="s")

    buf_set = [
        pltpu.VMEM((_G, _SUB), jnp.int32),
        pltpu.VMEM((_G, _SUB), jnp.int32),
        pltpu.VMEM((_G, _SUB), jnp.int32),
        pltpu.VMEM((_G * _SUB, d0), tab0.dtype),
        pltpu.VMEM((_G * _SUB, d1), tab1.dtype),
        pltpu.VMEM((_G * _SUB, d2), tab2.dtype),
        pltpu.SemaphoreType.DMA,
        pltpu.SemaphoreType.DMA,
        pltpu.SemaphoreType.DMA,
    ]

    @functools.partial(
        pl.kernel,
        out_type=jax.ShapeDtypeStruct((n_sub * _SUB, 128), tab0.dtype),
        mesh=mesh,
        scratch_types=buf_set * n_sets,
        compiler_params=pltpu.CompilerParams(use_tc_tiling_on_sc=False),
    )
    def k(i0_hbm, i1_hbm, i2_hbm, t0_hbm, t1_hbm, t2_hbm, o_hbm, *scratch):
        set_a, set_b = scratch[:9], scratch[9:] if pipelined else scratch[:9]
        wid = lax.axis_index("s") * _NC + lax.axis_index("c")
        tabs = (t0_hbm, t1_hbm, t2_hbm)
        idx_hbms = (i0_hbm, i1_hbm, i2_hbm)
        lane_off = (0, d0, d0 + d1)

        def s1(g, S):
            ivs, rvs, isem, gsem = S[0:3], S[3:6], S[6], S[7]
            base_sub = wid * per_w + g * _G
            for c in [pltpu.async_copy(ih.at[pl.ds(base_sub, _G)], iv, isem)
                      for ih, iv in zip(idx_hbms, ivs)]:
                c.wait()
            for j in range(_G):
                sl = pl.ds(j * _SUB, _SUB)
                for t, iv, rv in zip(tabs, ivs, rvs):
                    pltpu.async_copy(t.at[iv.at[j]], rv.at[sl], gsem)

        def s2(g, S):
            ivs, rvs, gsem, wsem = S[0:3], S[3:6], S[7], S[8]
            for j in range(_G):
                sl = pl.ds(j * _SUB, _SUB)
                for t, iv, rv in zip(tabs, ivs, rvs):
                    pltpu.make_async_copy(t.at[iv.at[j]], rv.at[sl], gsem).wait()
            rows = pl.ds((wid * per_w + g * _G) * _SUB, _G * _SUB)
            for rv, off, d in zip(rvs, lane_off, (d0, d1, d2)):
                pltpu.async_copy(rv, o_hbm.at[rows, pl.ds(off, d)], wsem)

        def s3(S):
            rvs, wsem = S[3:6], S[8]
            rows = pl.ds(0, _G * _SUB)
            for rv, off, d in zip(rvs, lane_off, (d0, d1, d2)):
                pltpu.make_async_copy(rv, o_hbm.at[rows, pl.ds(off, d)], wsem).wait()

        if pipelined:
            s1(0, set_a)
            s1(1, set_b)

            def body(p, carry):
                g = 2 * p
                s2(g, set_a)
                s2(g + 1, set_b)
                s3(set_a)
                s1(g + 2, set_a)
                s3(set_b)
                s1(g + 3, set_b)
                return carry

            lax.fori_loop(0, n_pair - 1, body, 0)
            s2(n_grp - 2, set_a)
            s2(n_grp - 1, set_b)
            s3(set_a)
            s3(set_b)
        else:
            def body(g, carry):
                s1(g, set_a)
                s2(g, set_a)
                s3(set_a)
                return carry

            lax.fori_loop(0, n_grp, body, 0)

    return k(idx0, idx1, idx2, tab0, tab1, tab2)


def _tc_body(comb_ref, w_ref, b_ref, o_ref, *, d_valid):
    x = comb_ref[...]
    lanes = lax.broadcasted_iota(jnp.int32, x.shape, 1)
    x = jnp.where(lanes < d_valid, x, 0.0)
    o_ref[...] = jnp.dot(x, w_ref[...],
                         preferred_element_type=jnp.float32) + b_ref[...]


def _tc_chunk_body(comb_ref, w_ref, b_ref, o_ref, *, d_valid):
    _tc_body(comb_ref, w_ref, b_ref, o_ref, d_valid=d_valid)


def _tc_chunk_body_alias(prev_ref, comb_ref, w_ref, b_ref, o_ref, *, d_valid):
    del prev_ref
    _tc_body(comb_ref, w_ref, b_ref, o_ref, d_valid=d_valid)


def _tc_combine_chunk(comb, w_pad, b, d_valid, n, chunk, prev_out):
    n_c = comb.shape[0]
    bs = 16384
    while n_c % bs:
        bs //= 2
    spc = n_c // bs
    blk = lambda i: (i, 0)
    out_blk = lambda i: (i + chunk * spc, 0)
    full = lambda i: (0, 0)
    d_out = w_pad.shape[1]
    out_shape = jax.ShapeDtypeStruct((n, d_out), jnp.float32)
    specs = [
        pl.BlockSpec((bs, comb.shape[1]), blk),
        pl.BlockSpec(w_pad.shape, full),
        pl.BlockSpec((1, d_out), full),
    ]
    body = functools.partial(_tc_body, d_valid=d_valid)
    if prev_out is None:
        return pl.pallas_call(
            body,
            grid=(spc,),
            in_specs=specs,
            out_specs=pl.BlockSpec((bs, d_out), out_blk),
            out_shape=out_shape,
        )(comb, w_pad, b)
    return pl.pallas_call(
        functools.partial(_tc_chunk_body_alias, d_valid=d_valid),
        grid=(spc,),
        in_specs=[pl.BlockSpec((8, d_out), full)] + specs,
        out_specs=pl.BlockSpec((bs, d_out), out_blk),
        out_shape=out_shape,
        input_output_aliases={0: 0},
    )(prev_out, comb, w_pad, b)


def kernel(card_ids, mana_costs, card_types, powers, toughnesses,
           card_table, mana_table, type_table, power_table, tough_table, W, b):
    bsz, seq = card_ids.shape
    n = bsz * seq
    n_mana = mana_table.shape[0]
    n_type = type_table.shape[0]
    n_pow = power_table.shape[0]
    n_tgh = tough_table.shape[0]

    mt_tab = jnp.concatenate(
        [jnp.repeat(mana_table, n_type, axis=0), jnp.tile(type_table, (n_mana, 1))],
        axis=1)
    pt_tab = jnp.concatenate(
        [jnp.repeat(power_table, n_tgh, axis=0), jnp.tile(tough_table, (n_pow, 1))],
        axis=1)

    to2d = lambda a: a.reshape(n // _SUB, _SUB).astype(jnp.int32)
    card_idx = to2d(card_ids)
    mt_idx = to2d(mana_costs * n_type + card_types)
    pt_idx = to2d(powers * n_tgh + toughnesses)

    w_pad = jnp.concatenate(
        [W, jnp.zeros((128 - W.shape[0], W.shape[1]), W.dtype)], axis=0)
    b2 = b.reshape(1, -1)

    n_chunks = 1
    n_sub_c = (n // _SUB) // n_chunks
    out = None
    for c in range(n_chunks):
        sl = slice(c * n_sub_c, (c + 1) * n_sub_c)
        comb_c = _sc_gather3(card_idx[sl], mt_idx[sl], pt_idx[sl],
                             card_table, mt_tab, pt_tab)
        out = _tc_combine_chunk(comb_c, w_pad, b2, W.shape[0], n, c, out)
    return out.reshape(bsz, seq, W.shape[1])
